# baseline (speedup 1.0000x reference)
import jax
import jax.numpy as jnp
from jax import lax
from jax.experimental import pallas as pl
from jax.experimental.pallas import tpu as pltpu
from jax.experimental.pallas import tpu_sc as plsc

N = 50000
E = 800000
D = 64
NG = 256
NW = 32
CK = 512
SLAB = 8 * CK
NSB = -(-E // SLAB)
G2 = 8 * NSB
EPAD = G2 * CK
R = 1568
BLK = 2000
NB = N // BLK
BLK2 = 10000
NB2 = N // BLK2
EPS = 1e-5


TSUP = 7


def _transpose_body(x_ref, o_ref):
    o_ref[...] = x_ref[...].T.reshape(D, 8 * TSUP, CK)


_transpose = pl.pallas_call(
    _transpose_body,
    grid=(NSB // TSUP,),
    in_specs=[pl.BlockSpec((TSUP * SLAB, D), lambda i: (i, 0))],
    out_specs=pl.BlockSpec((D, 8 * TSUP, CK), lambda i: (0, i, 0)),
    out_shape=jax.ShapeDtypeStruct((D, G2, CK), jnp.float32),
)


def _sc_body(eaT, srcv, sumsT, counts, acc0, acc1, cacc,
             eb00, eb01, eb10, eb11, ib0, ib1, sem0, sem1):
    c = lax.axis_index("c")
    s = lax.axis_index("s")
    w = s * 2 + c
    col0 = 2 * w
    base = w * R

    zf = jnp.zeros((16,), jnp.float32)

    def zacc(i, _):
        acc0[pl.ds(i * 16, 16)] = zf
        acc1[pl.ds(i * 16, 16)] = zf
        return 0
    lax.fori_loop(0, N // 16, zacc, 0)

    def zcnt(i, _):
        cacc[pl.ds(i * 16, 16)] = zf
        return 0
    lax.fori_loop(0, R // 16, zcnt, 0)

    sems = (sem0, sem1)
    ebufs = ((eb00, eb01), (eb10, eb11))
    ibufs = (ib0, ib1)

    def chunk8(q):
        return pl.ds(pl.multiple_of(8 * q, 8), 8)

    def issue(q, b):
        pltpu.async_copy(eaT.at[col0, chunk8(q), :], ebufs[b][0], sems[b])
        pltpu.async_copy(eaT.at[col0 + 1, chunk8(q), :], ebufs[b][1], sems[b])
        pltpu.async_copy(srcv.at[pl.ds(q * SLAB, SLAB)], ibufs[b], sems[b])

    def wait(q, b):
        pltpu.make_async_copy(eaT.at[col0, chunk8(q), :], ebufs[b][0], sems[b]).wait()
        pltpu.make_async_copy(eaT.at[col0 + 1, chunk8(q), :], ebufs[b][1], sems[b]).wait()
        pltpu.make_async_copy(srcv.at[pl.ds(q * SLAB, SLAB)], ibufs[b], sems[b]).wait()

    issue(0, 0)
    issue(1, 1)

    ones16 = jnp.ones((16,), jnp.float32)

    def process(q, b, padded):
        wait(q, b)
        ib = ibufs[b]
        vb0, vb1 = ebufs[b]
        for r in range(8):
            def step(j):
                o = j * 16
                iv = ib[pl.ds(r * CK + o, 16)]
                v0 = vb0[r, pl.ds(o, 16)]
                v1 = vb1[r, pl.ds(o, 16)]
                if padded:
                    mv = iv >= 0
                    ivc = jnp.maximum(iv, 0)
                    plsc.addupdate_scatter(acc0, [ivc], v0, mask=mv)
                    plsc.addupdate_scatter(acc1, [ivc], v1, mask=mv)
                else:
                    plsc.addupdate_scatter(acc0, [iv], v0)
                    plsc.addupdate_scatter(acc1, [iv], v1)
                rel = iv - base
                m = (rel >= 0) & (rel < R)
                relc = jnp.where(m, rel, 0)
                plsc.addupdate_scatter(cacc, [relc], ones16, mask=m)
            plsc.parallel_loop(0, CK // 16, 1, unroll=4, carry=None)(
                lambda j, _=None: step(j))

    def outer(p, _):
        q = p * 2
        process(q, 0, False)
        issue(q + 2, 0)
        process(q + 1, 1, False)
        issue(q + 3, 1)
        return 0
    lax.fori_loop(0, NSB // 2 - 1, outer, 0)
    process(NSB - 2, 0, False)
    process(NSB - 1, 1, True)

    def wout(ib2, _):
        pltpu.sync_copy(acc0.at[pl.ds(ib2 * BLK, BLK)],
                        sumsT.at[pl.ds(pl.multiple_of((ib2 * D + col0) * BLK, 8), BLK)])
        pltpu.sync_copy(acc1.at[pl.ds(ib2 * BLK, BLK)],
                        sumsT.at[pl.ds(pl.multiple_of((ib2 * D + col0 + 1) * BLK, 8), BLK)])
        return 0
    lax.fori_loop(0, NB, wout, 0)
    pltpu.sync_copy(cacc, counts.at[pl.ds(pl.multiple_of(w * R, 8), R)])


_sc_scatter = pl.kernel(
    _sc_body,
    out_type=(
        jax.ShapeDtypeStruct((NB * D * BLK,), jnp.float32),
        jax.ShapeDtypeStruct((NW * R,), jnp.float32),
    ),
    mesh=plsc.VectorSubcoreMesh(core_axis_name="c", subcore_axis_name="s"),
    scratch_types=[
        pltpu.VMEM((N,), jnp.float32),
        pltpu.VMEM((N,), jnp.float32),
        pltpu.VMEM((R,), jnp.float32),
        pltpu.VMEM((8, CK), jnp.float32),
        pltpu.VMEM((8, CK), jnp.float32),
        pltpu.VMEM((8, CK), jnp.float32),
        pltpu.VMEM((8, CK), jnp.float32),
        pltpu.VMEM((SLAB,), jnp.int32),
        pltpu.VMEM((SLAB,), jnp.int32),
        pltpu.SemaphoreType.DMA,
        pltpu.SemaphoreType.DMA,
    ],
    compiler_params=pltpu.CompilerParams(needs_layout_passes=False),
)


def _layer1_body(x_ref, sT_ref, cnt_ref, bat_ref, ga_ref, w_ref, b_ref,
                 a_ref, st_ref):
    i = pl.program_id(0)
    cnt = cnt_ref[0]
    inv = 1.0 / jnp.maximum(cnt, 1.0)
    veT = sT_ref[0] * inv
    bat = bat_ref[0]
    oh = (lax.broadcasted_iota(jnp.int32, (NG, BLK), 0) == bat).astype(jnp.float32)
    gb = lax.dot_general(oh, ga_ref[...], (((0,), (0,)), ((), ())),
                         preferred_element_type=jnp.float32)
    w = w_ref[...]
    z = jnp.dot(x_ref[...], w[0:D], preferred_element_type=jnp.float32)
    z += lax.dot_general(veT, w[D:2 * D], (((0,), (0,)), ((), ())),
                         preferred_element_type=jnp.float32)
    z += jnp.dot(gb, w[2 * D:3 * D], preferred_element_type=jnp.float32)
    z += b_ref[...]
    a = jnp.maximum(z, 0.0)
    a_ref[...] = a

    @pl.when(i == 0)
    def _():
        st_ref[...] = jnp.zeros_like(st_ref)
    su = jnp.sum(a, axis=0)
    sq = jnp.sum(a * a, axis=0)
    st_ref[...] += jnp.concatenate([su[None, :], sq[None, :]], axis=0)


def _mid_body(a_ref, sc_ref, sh_ref, w_ref, b_ref, o_ref, st_ref):
    i = pl.program_id(0)
    h = a_ref[...] * sc_ref[...] + sh_ref[...]
    z = jnp.dot(h, w_ref[...], preferred_element_type=jnp.float32) + b_ref[...]
    a = jnp.maximum(z, 0.0)
    o_ref[...] = a

    @pl.when(i == 0)
    def _():
        st_ref[...] = jnp.zeros_like(st_ref)
    su = jnp.sum(a, axis=0)
    sq = jnp.sum(a * a, axis=0)
    st_ref[...] += jnp.concatenate([su[None, :], sq[None, :]], axis=0)


def _final_body(a_ref, sc_ref, sh_ref, o_ref):
    o_ref[...] = a_ref[...] * sc_ref[...] + sh_ref[...]


_row_spec = pl.BlockSpec((BLK, D), lambda i: (i, 0))
_row2_spec = pl.BlockSpec((BLK2, D), lambda i: (i, 0))
_vec_spec = pl.BlockSpec((1, D), lambda i: (0, 0))
_st_spec = pl.BlockSpec((2, D), lambda i: (0, 0))

_layer1 = pl.pallas_call(
    _layer1_body,
    grid=(NB,),
    in_specs=[
        _row_spec,
        pl.BlockSpec((1, D, BLK), lambda i: (i, 0, 0)),
        pl.BlockSpec((1, 1, BLK), lambda i: (i, 0, 0)),
        pl.BlockSpec((1, 1, BLK), lambda i: (i, 0, 0)),
        pl.BlockSpec((NG, D), lambda i: (0, 0)),
        pl.BlockSpec((3 * D, D), lambda i: (0, 0)),
        _vec_spec,
    ],
    out_specs=[_row_spec, _st_spec],
    out_shape=[
        jax.ShapeDtypeStruct((N, D), jnp.float32),
        jax.ShapeDtypeStruct((2, D), jnp.float32),
    ],
)

_mid = pl.pallas_call(
    _mid_body,
    grid=(NB2,),
    in_specs=[
        _row2_spec,
        _vec_spec,
        _vec_spec,
        pl.BlockSpec((D, D), lambda i: (0, 0)),
        _vec_spec,
    ],
    out_specs=[_row2_spec, _st_spec],
    out_shape=[
        jax.ShapeDtypeStruct((N, D), jnp.float32),
        jax.ShapeDtypeStruct((2, D), jnp.float32),
    ],
)

_final = pl.pallas_call(
    _final_body,
    grid=(NB2,),
    in_specs=[_row2_spec, _vec_spec, _vec_spec],
    out_specs=_row2_spec,
    out_shape=jax.ShapeDtypeStruct((N, D), jnp.float32),
)


def _affine(st, gamma, beta):
    m = st[0] / N
    v = st[1] / N - m * m
    s = gamma / jnp.sqrt(v + EPS)
    t = beta - m * s
    return s.reshape(1, D), t.reshape(1, D)


def kernel(x, edge_index, edge_attr, global_attr, batch,
           W1, b1, W2, b2, W3, b3, g1, be1, g2, be2, g3, be3):
    src = jnp.pad(edge_index[0].astype(jnp.int32), (0, EPAD - E),
                  constant_values=-1)
    eaT = _transpose(edge_attr)
    sums_f, counts = _sc_scatter(eaT, src)
    sumsT = sums_f.reshape(NB, D, BLK)
    cnt3 = counts[:N].reshape(NB, 1, BLK)
    bat3 = batch.astype(jnp.int32).reshape(NB, 1, BLK)

    a1, st1 = _layer1(x, sumsT, cnt3, bat3, global_attr, W1, b1.reshape(1, D))
    s1, t1 = _affine(st1, g1, be1)
    a2, st2 = _mid(a1, s1, t1, W2, b2.reshape(1, D))
    s2, t2 = _affine(st2, g2, be2)
    a3, st3 = _mid(a2, s2, t2, W3, b3.reshape(1, D))
    s3, t3 = _affine(st3, g3, be3)
    return _final(a3, s3, t3)


# speedup vs baseline: 1.0045x; 1.0045x over previous
import jax
import jax.numpy as jnp
from jax import lax
from jax.experimental import pallas as pl
from jax.experimental.pallas import tpu as pltpu
from jax.experimental.pallas import tpu_sc as plsc

N = 50000
E = 800000
D = 64
NG = 256
NW = 32
CK = 512
SLAB = 8 * CK
NSB = -(-E // SLAB)
G2 = 8 * NSB
EPAD = G2 * CK
R = 1568
BLK = 2000
NB = N // BLK
BLK2 = 10000
NB2 = N // BLK2
EPS = 1e-5


TSUP = 7
HSB = NSB // 2
HG2 = 8 * HSB
HE = HSB * SLAB


def _transpose_body(x_ref, o_ref):
    o_ref[...] = x_ref[...].T.reshape(D, 8 * TSUP, CK)


def _make_transpose(half):
    return pl.pallas_call(
        _transpose_body,
        grid=(HSB // TSUP,),
        in_specs=[pl.BlockSpec((TSUP * SLAB, D),
                               lambda i: (i + half * (HSB // TSUP), 0))],
        out_specs=pl.BlockSpec((D, 8 * TSUP, CK), lambda i: (0, i, 0)),
        out_shape=jax.ShapeDtypeStruct((D, HG2, CK), jnp.float32),
    )


_transpose_a = _make_transpose(0)
_transpose_b = _make_transpose(1)


def _sc_body(eaT, srcv, sumsT, counts, acc0, acc1, cacc,
             eb00, eb01, eb10, eb11, ib0, ib1, sem0, sem1,
             nsb=HSB, has_pad=False):
    c = lax.axis_index("c")
    s = lax.axis_index("s")
    w = s * 2 + c
    col0 = 2 * w
    base = w * R

    zf = jnp.zeros((16,), jnp.float32)

    def zacc(i, _):
        acc0[pl.ds(i * 16, 16)] = zf
        acc1[pl.ds(i * 16, 16)] = zf
        return 0
    lax.fori_loop(0, N // 16, zacc, 0)

    def zcnt(i, _):
        cacc[pl.ds(i * 16, 16)] = zf
        return 0
    lax.fori_loop(0, R // 16, zcnt, 0)

    sems = (sem0, sem1)
    ebufs = ((eb00, eb01), (eb10, eb11))
    ibufs = (ib0, ib1)

    def chunk8(q):
        return pl.ds(pl.multiple_of(8 * q, 8), 8)

    def issue(q, b):
        pltpu.async_copy(eaT.at[col0, chunk8(q), :], ebufs[b][0], sems[b])
        pltpu.async_copy(eaT.at[col0 + 1, chunk8(q), :], ebufs[b][1], sems[b])
        pltpu.async_copy(srcv.at[pl.ds(q * SLAB, SLAB)], ibufs[b], sems[b])

    def wait(q, b):
        pltpu.make_async_copy(eaT.at[col0, chunk8(q), :], ebufs[b][0], sems[b]).wait()
        pltpu.make_async_copy(eaT.at[col0 + 1, chunk8(q), :], ebufs[b][1], sems[b]).wait()
        pltpu.make_async_copy(srcv.at[pl.ds(q * SLAB, SLAB)], ibufs[b], sems[b]).wait()

    issue(0, 0)
    issue(1, 1)

    ones16 = jnp.ones((16,), jnp.float32)

    def process(q, b, padded):
        wait(q, b)
        ib = ibufs[b]
        vb0, vb1 = ebufs[b]
        for r in range(8):
            def step(j):
                o = j * 16
                iv = ib[pl.ds(r * CK + o, 16)]
                v0 = vb0[r, pl.ds(o, 16)]
                v1 = vb1[r, pl.ds(o, 16)]
                if padded:
                    mv = iv >= 0
                    ivc = jnp.maximum(iv, 0)
                    plsc.addupdate_scatter(acc0, [ivc], v0, mask=mv)
                    plsc.addupdate_scatter(acc1, [ivc], v1, mask=mv)
                else:
                    plsc.addupdate_scatter(acc0, [iv], v0)
                    plsc.addupdate_scatter(acc1, [iv], v1)
                rel = iv - base
                m = (rel >= 0) & (rel < R)
                relc = jnp.where(m, rel, 0)
                plsc.addupdate_scatter(cacc, [relc], ones16, mask=m)
            plsc.parallel_loop(0, CK // 16, 1, unroll=4, carry=None)(
                lambda j, _=None: step(j))

    def outer(p, _):
        q = p * 2
        process(q, 0, False)
        issue(q + 2, 0)
        process(q + 1, 1, False)
        issue(q + 3, 1)
        return 0
    lax.fori_loop(0, nsb // 2 - 1, outer, 0)
    process(nsb - 2, 0, False)
    process(nsb - 1, 1, has_pad)

    def wout(ib2, _):
        pltpu.sync_copy(acc0.at[pl.ds(ib2 * BLK, BLK)],
                        sumsT.at[pl.ds(pl.multiple_of((ib2 * D + col0) * BLK, 8), BLK)])
        pltpu.sync_copy(acc1.at[pl.ds(ib2 * BLK, BLK)],
                        sumsT.at[pl.ds(pl.multiple_of((ib2 * D + col0 + 1) * BLK, 8), BLK)])
        return 0
    lax.fori_loop(0, NB, wout, 0)
    pltpu.sync_copy(cacc, counts.at[pl.ds(pl.multiple_of(w * R, 8), R)])


def _make_sc(has_pad):
    def body(*refs):
        _sc_body(*refs, nsb=HSB, has_pad=has_pad)
    return pl.kernel(
        body,
        out_type=(
            jax.ShapeDtypeStruct((NB * D * BLK,), jnp.float32),
            jax.ShapeDtypeStruct((NW * R,), jnp.float32),
        ),
        mesh=plsc.VectorSubcoreMesh(core_axis_name="c", subcore_axis_name="s"),
        scratch_types=[
        pltpu.VMEM((N,), jnp.float32),
        pltpu.VMEM((N,), jnp.float32),
        pltpu.VMEM((R,), jnp.float32),
        pltpu.VMEM((8, CK), jnp.float32),
        pltpu.VMEM((8, CK), jnp.float32),
        pltpu.VMEM((8, CK), jnp.float32),
        pltpu.VMEM((8, CK), jnp.float32),
            pltpu.VMEM((SLAB,), jnp.int32),
            pltpu.VMEM((SLAB,), jnp.int32),
            pltpu.SemaphoreType.DMA,
            pltpu.SemaphoreType.DMA,
        ],
        compiler_params=pltpu.CompilerParams(needs_layout_passes=False),
    )


_sc_a = _make_sc(False)
_sc_b = _make_sc(True)


def _layer1_body(x_ref, sTa_ref, sTb_ref, cnta_ref, cntb_ref, bat_ref,
                 ga_ref, w_ref, b_ref, a_ref, st_ref):
    i = pl.program_id(0)
    cnt = cnta_ref[0] + cntb_ref[0]
    inv = 1.0 / jnp.maximum(cnt, 1.0)
    veT = (sTa_ref[0] + sTb_ref[0]) * inv
    bat = bat_ref[0]
    oh = (lax.broadcasted_iota(jnp.int32, (NG, BLK), 0) == bat).astype(jnp.float32)
    gb = lax.dot_general(oh, ga_ref[...], (((0,), (0,)), ((), ())),
                         preferred_element_type=jnp.float32)
    w = w_ref[...]
    z = jnp.dot(x_ref[...], w[0:D], preferred_element_type=jnp.float32)
    z += lax.dot_general(veT, w[D:2 * D], (((0,), (0,)), ((), ())),
                         preferred_element_type=jnp.float32)
    z += jnp.dot(gb, w[2 * D:3 * D], preferred_element_type=jnp.float32)
    z += b_ref[...]
    a = jnp.maximum(z, 0.0)
    a_ref[...] = a

    @pl.when(i == 0)
    def _():
        st_ref[...] = jnp.zeros_like(st_ref)
    su = jnp.sum(a, axis=0)
    sq = jnp.sum(a * a, axis=0)
    st_ref[...] += jnp.concatenate([su[None, :], sq[None, :]], axis=0)


def _mid_body(a_ref, sc_ref, sh_ref, w_ref, b_ref, o_ref, st_ref):
    i = pl.program_id(0)
    h = a_ref[...] * sc_ref[...] + sh_ref[...]
    z = jnp.dot(h, w_ref[...], preferred_element_type=jnp.float32) + b_ref[...]
    a = jnp.maximum(z, 0.0)
    o_ref[...] = a

    @pl.when(i == 0)
    def _():
        st_ref[...] = jnp.zeros_like(st_ref)
    su = jnp.sum(a, axis=0)
    sq = jnp.sum(a * a, axis=0)
    st_ref[...] += jnp.concatenate([su[None, :], sq[None, :]], axis=0)


def _final_body(a_ref, sc_ref, sh_ref, o_ref):
    o_ref[...] = a_ref[...] * sc_ref[...] + sh_ref[...]


_row_spec = pl.BlockSpec((BLK, D), lambda i: (i, 0))
_row2_spec = pl.BlockSpec((BLK2, D), lambda i: (i, 0))
_vec_spec = pl.BlockSpec((1, D), lambda i: (0, 0))
_st_spec = pl.BlockSpec((2, D), lambda i: (0, 0))

_layer1 = pl.pallas_call(
    _layer1_body,
    grid=(NB,),
    in_specs=[
        _row_spec,
        pl.BlockSpec((1, D, BLK), lambda i: (i, 0, 0)),
        pl.BlockSpec((1, D, BLK), lambda i: (i, 0, 0)),
        pl.BlockSpec((1, 1, BLK), lambda i: (i, 0, 0)),
        pl.BlockSpec((1, 1, BLK), lambda i: (i, 0, 0)),
        pl.BlockSpec((1, 1, BLK), lambda i: (i, 0, 0)),
        pl.BlockSpec((NG, D), lambda i: (0, 0)),
        pl.BlockSpec((3 * D, D), lambda i: (0, 0)),
        _vec_spec,
    ],
    out_specs=[_row_spec, _st_spec],
    out_shape=[
        jax.ShapeDtypeStruct((N, D), jnp.float32),
        jax.ShapeDtypeStruct((2, D), jnp.float32),
    ],
)

_mid = pl.pallas_call(
    _mid_body,
    grid=(NB2,),
    in_specs=[
        _row2_spec,
        _vec_spec,
        _vec_spec,
        pl.BlockSpec((D, D), lambda i: (0, 0)),
        _vec_spec,
    ],
    out_specs=[_row2_spec, _st_spec],
    out_shape=[
        jax.ShapeDtypeStruct((N, D), jnp.float32),
        jax.ShapeDtypeStruct((2, D), jnp.float32),
    ],
)

_final = pl.pallas_call(
    _final_body,
    grid=(NB2,),
    in_specs=[_row2_spec, _vec_spec, _vec_spec],
    out_specs=_row2_spec,
    out_shape=jax.ShapeDtypeStruct((N, D), jnp.float32),
)


def _affine(st, gamma, beta):
    m = st[0] / N
    v = st[1] / N - m * m
    s = gamma / jnp.sqrt(v + EPS)
    t = beta - m * s
    return s.reshape(1, D), t.reshape(1, D)


def kernel(x, edge_index, edge_attr, global_attr, batch,
           W1, b1, W2, b2, W3, b3, g1, be1, g2, be2, g3, be3):
    src = jnp.pad(edge_index[0].astype(jnp.int32), (0, EPAD - E),
                  constant_values=-1)
    src_a = lax.slice(src, (0,), (HE,))
    src_b = lax.slice(src, (HE,), (EPAD,))
    ta = _transpose_a(edge_attr)
    sa_f, ca = _sc_a(ta, src_a)
    tb = _transpose_b(edge_attr)
    sb_f, cb = _sc_b(tb, src_b)
    sumsTa = sa_f.reshape(NB, D, BLK)
    sumsTb = sb_f.reshape(NB, D, BLK)
    cnt3a = ca[:N].reshape(NB, 1, BLK)
    cnt3b = cb[:N].reshape(NB, 1, BLK)
    bat3 = batch.astype(jnp.int32).reshape(NB, 1, BLK)

    a1, st1 = _layer1(x, sumsTa, sumsTb, cnt3a, cnt3b, bat3, global_attr,
                      W1, b1.reshape(1, D))
    s1, t1 = _affine(st1, g1, be1)
    a2, st2 = _mid(a1, s1, t1, W2, b2.reshape(1, D))
    s2, t2 = _affine(st2, g2, be2)
    a3, st3 = _mid(a2, s2, t2, W3, b3.reshape(1, D))
    s3, t3 = _affine(st3, g3, be3)
    return _final(a3, s3, t3)
